# Initial kernel scaffold; baseline (speedup 1.0000x reference)
import jax
import jax.numpy as jnp
from jax.experimental import pallas as pl


def kernel(node_attr, edge_index, edge_attr, edge_sh, W_lin_in, fc_W1, fc_b1, fc_W2, fc_b2, fc_W3, fc_b3, W_lin_out):
    raise NotImplementedError("write your pallas kernel here")


# speedup vs baseline: 12.9730x; 12.9730x over previous
import functools

import numpy as np
import jax
import jax.numpy as jnp
from jax import lax
from jax.experimental import pallas as pl
from jax.experimental.pallas import tpu as pltpu
from jax.experimental.pallas import tpu_sc as plsc

F32 = jnp.float32


def _blkdiag(w, p):
    a, b = w.shape
    out = jnp.zeros((p * a, p * b), w.dtype)
    for j in range(p):
        out = out.at[j * a:(j + 1) * a, j * b:(j + 1) * b].set(w)
    return out


def _lin_in_body(x_ref, w_ref, o_ref):
    o_ref[...] = jnp.dot(x_ref[...], w_ref[...], preferred_element_type=F32)


def _edge_body(ea_ref, sh_ref, xs_ref, w1_ref, b1_ref, w2_ref, b2_ref,
               w3_ref, b3_ref, rsh_ref, rrep_ref, ssum_ref, o_ref):
    bf = jnp.bfloat16
    br8 = xs_ref.shape[0]
    h = jnp.maximum(jnp.dot(ea_ref[...].astype(bf), w1_ref[...],
                            preferred_element_type=F32) + b1_ref[...], 0.0)
    h = jnp.maximum(jnp.dot(h.astype(bf), w2_ref[...],
                            preferred_element_type=F32) + b2_ref[...], 0.0)
    w = jnp.dot(h.astype(bf), w3_ref[...],
                preferred_element_type=F32) + b3_ref[...]
    xs = xs_ref[...].reshape(br8 * 8, 128)[:, :64]
    a = xs * jnp.dot(sh_ref[...].astype(bf), rsh_ref[...],
                     preferred_element_type=F32)
    prod = jnp.dot(a.astype(bf), rrep_ref[...],
                   preferred_element_type=F32) * w
    tp = jnp.dot(prod.astype(bf), ssum_ref[...],
                 preferred_element_type=F32)
    o_ref[:, :, :64] = tp.reshape(br8, 8, 64)


def _out_body(ps_ref, pc_ref, na_ref, w_ref, rc_ref, o_ref):
    s = ps_ref[0] + ps_ref[1]
    c = jnp.maximum(pc_ref[0] + pc_ref[1], 1.0)
    cexp = jnp.dot(c, rc_ref[...], preferred_element_type=F32)
    o_ref[...] = jnp.dot(s, w_ref[...], preferred_element_type=F32) / cexp \
        + na_ref[...]


def kernel(node_attr, edge_index, edge_attr, edge_sh, W_lin_in, fc_W1, fc_b1,
           fc_W2, fc_b2, fc_W3, fc_b3, W_lin_out):
    N, D_IN = node_attr.shape
    E = edge_index.shape[1]
    N_EDGE = edge_attr.shape[1]
    D_TP_IN = W_lin_in.shape[1]
    FC = fc_W1.shape[1]
    D_TP_OUT = W_lin_out.shape[0]
    D_OUT = W_lin_out.shape[1]
    P = 8
    RP = E // P

    na_p = node_attr.reshape(N // 8, 8 * D_IN)
    big_win = _blkdiag(W_lin_in * (1.0 / np.sqrt(D_IN)), 8)
    table_p = pl.pallas_call(
        _lin_in_body,
        out_shape=jax.ShapeDtypeStruct((N // 8, 8 * D_TP_IN), F32),
    )(na_p, big_win)
    table = table_p.reshape(N, D_TP_IN)

    src = edge_index[0]
    dst = edge_index[1]
    NW = 32
    EW = E // NW
    CH = 2000
    NCH = EW // CH
    RT = N // 16
    mesh = plsc.VectorSubcoreMesh(core_axis_name="c", subcore_axis_name="s")

    ea_flat = edge_attr.reshape(E // 128, 128, N_EDGE)\
        .transpose(0, 2, 1).reshape(E * N_EDGE)
    NBLK = E // 128
    EABLK = CH // 128 + 2
    CHR = CH // P

    XSROWS = (RP // 8) * 8 * 128 // D_TP_IN

    def _tile_row_idx(i, off, idx2_v):
        e = off + i * 16 + lax.iota(jnp.int32, 16)
        v = ((e >> 6) << 7) + (((e >> 3) & 7) << 4) + (e & 7)
        idx2_v[pl.ds(i * 16, 16)] = v

    @functools.partial(
        pl.kernel,
        out_type=[
            jax.ShapeDtypeStruct((XSROWS, D_TP_IN), F32),
            jax.ShapeDtypeStruct((RP * P * N_EDGE,), F32),
        ],
        mesh=mesh,
        compiler_params=pltpu.CompilerParams(use_tc_tiling_on_sc=False,
                                             needs_layout_passes=False),
        scratch_types=[
            pltpu.VMEM((CH,), jnp.int32),
            pltpu.VMEM((CH,), jnp.int32),
            pltpu.VMEM((CH,), jnp.int32),
            pltpu.VMEM((CH,), jnp.int32),
            pltpu.VMEM((CH, D_TP_IN), F32),
            pltpu.VMEM((CH, D_TP_IN), F32),
            pltpu.VMEM((EABLK * 512,), F32),
            pltpu.VMEM((CHR * P * N_EDGE,), F32),
            pltpu.VMEM_SHARED((N, D_TP_IN), F32),
            pltpu.SemaphoreType.DMA,
            pltpu.SemaphoreType.DMA,
            pltpu.SemaphoreType.DMA,
            pltpu.SemaphoreType.DMA,
        ],
    )
    def _gather_k(table_hbm, src_hbm, eat_hbm, xs_hbm, eaf_hbm,
                  idx0, idx1, t0, t1, rows0, rows1, eas_v, eaf_v,
                  tab_sh, gs0, gs1, ws0, ws1):
        c = lax.axis_index("c")
        s = lax.axis_index("s")
        pltpu.sync_copy(table_hbm.at[pl.ds(s * RT, RT)],
                        tab_sh.at[pl.ds(s * RT, RT)])
        plsc.subcore_barrier()
        base = (s * 2 + c) * EW
        baser = base // P
        idx = (idx0, idx1)
        tid = (t0, t1)
        rows = (rows0, rows1)
        gsem = (gs0, gs1)
        wsem = (ws0, ws1)
        pltpu.sync_copy(src_hbm.at[pl.ds(base, CH)], idx[0])
        lax.fori_loop(0, CH // 16,
                      lambda i, _: (_tile_row_idx(i, base, tid[0]), 0)[1],
                      0, unroll=4)
        g = {0: pltpu.async_copy(tab_sh.at[idx[0]], rows[0], gsem[0])}
        w = {}
        for k in range(NCH):
            b = k % 2
            nb = (k + 1) % 2
            if k + 1 < NCH:
                pltpu.sync_copy(src_hbm.at[pl.ds(base + (k + 1) * CH, CH)],
                                idx[nb])
            offr = baser + k * CHR
            blk0 = jnp.minimum((base + k * CH) // 128, NBLK - EABLK)
            pltpu.sync_copy(eat_hbm.at[pl.ds(blk0 * 512, EABLK * 512)],
                            eas_v)
            io16 = lax.iota(jnp.int32, 16)
            pat = ((io16 >> 3) << 7) + (io16 & 7)

            def _repack(i, _):
                q = 8 * (baser + k * CHR + i) - blk0 * 128
                p0 = ((q >> 7) << 9) + (q & 127)
                idxv = p0 + pat
                v = plsc.load_gather(eas_v, [idxv])
                eaf_v[pl.ds(i * 32, 16)] = v
                v2 = plsc.load_gather(eas_v, [idxv + 256])
                eaf_v[pl.ds(i * 32 + 16, 16)] = v2
                return 0

            lax.fori_loop(0, CHR, _repack, 0)
            pltpu.sync_copy(eaf_v, eaf_hbm.at[pl.ds(offr * 32, CHR * 32)])
            g[k].wait()
            if k + 1 < NCH:
                if k >= 1:
                    w[k - 1].wait()
                lax.fori_loop(
                    0, CH // 16,
                    lambda i, _: (_tile_row_idx(i, base + (k + 1) * CH,
                                                tid[nb]), 0)[1],
                    0, unroll=4)
                g[k + 1] = pltpu.async_copy(tab_sh.at[idx[nb]], rows[nb],
                                            gsem[nb])
            w[k] = pltpu.async_copy(rows[b], xs_hbm.at[tid[b]], wsem[b])
        w[NCH - 2].wait()
        w[NCH - 1].wait()

    xs, ea_f = _gather_k(table, src, ea_flat)
    ea_f = ea_f.reshape(RP, P * N_EDGE)

    sh_p = edge_sh.reshape(RP, P)

    tp_norm = 1.0 / np.sqrt(D_TP_IN * 1)
    big_w1 = jnp.zeros((P * N_EDGE, P * FC), F32)
    for j in range(P):
        for f in range(N_EDGE):
            big_w1 = big_w1.at[f * P + j, j * FC:(j + 1) * FC].set(fc_W1[f])
    big_b1 = jnp.tile(fc_b1, P).reshape(1, P * FC)
    big_w2 = _blkdiag(fc_W2, P)
    big_b2 = jnp.tile(fc_b2, P).reshape(1, P * FC)
    big_w3 = _blkdiag(fc_W3 * tp_norm, P)
    big_b3 = jnp.tile(fc_b3 * tp_norm, P).reshape(1, P * D_TP_IN * D_TP_OUT)

    WN = D_TP_IN * D_TP_OUT
    rsh = np.zeros((P, P * D_TP_IN), np.float32)
    for j in range(P):
        rsh[j, j * D_TP_IN:(j + 1) * D_TP_IN] = 1.0
    rrep = np.zeros((P * D_TP_IN, P * WN), np.float32)
    ssum = np.zeros((P * WN, P * D_TP_OUT), np.float32)
    for j in range(P):
        for i in range(D_TP_IN):
            for k in range(D_TP_OUT):
                rrep[j * D_TP_IN + i, j * WN + i * D_TP_OUT + k] = 1.0
                ssum[j * WN + i * D_TP_OUT + k, j * D_TP_OUT + k] = 1.0
    rsh = jnp.asarray(rsh)
    rrep = jnp.asarray(rrep)
    ssum = jnp.asarray(ssum)

    BR = 2000
    grid = (RP // BR,)
    full = lambda r, c_: pl.BlockSpec((r, c_), lambda i: (0, 0))
    xs_t = xs.reshape(RP // 8, 8, 128)
    tp_t = pl.pallas_call(
        _edge_body,
        grid=grid,
        in_specs=[
            pl.BlockSpec((BR, P * N_EDGE), lambda i: (i, 0)),
            pl.BlockSpec((BR, P), lambda i: (i, 0)),
            pl.BlockSpec((BR // 8, 8, 128), lambda i: (i, 0, 0)),
            full(P * N_EDGE, P * FC), full(1, P * FC),
            full(P * FC, P * FC), full(1, P * FC),
            full(P * FC, P * WN), full(1, P * WN),
            full(P, P * D_TP_IN),
            full(P * D_TP_IN, P * WN),
            full(P * WN, P * D_TP_OUT),
        ],
        out_specs=pl.BlockSpec((BR // 8, 8, 128), lambda i: (i, 0, 0)),
        out_shape=jax.ShapeDtypeStruct((RP // 8, 8, 128), F32),
    )(ea_f, sh_p, xs_t, big_w1, big_b1, big_w2, big_b2, big_w3, big_b3,
      rsh, rrep, ssum)
    tp_rows = tp_t.reshape(XSROWS, D_TP_OUT)

    NT = 16
    RT = N // NT
    CQ = N // 4
    CHD = 2000
    NCHD = EW // CHD
    z8 = jnp.zeros((N, D_TP_OUT), F32)
    z1 = jnp.zeros((N,), F32)
    ones_h = jnp.ones((CHD,), F32)

    @functools.partial(
        pl.kernel,
        out_type=[
            jax.ShapeDtypeStruct((2, N, D_TP_OUT), F32),
            jax.ShapeDtypeStruct((2, N), F32),
        ],
        mesh=mesh,
        compiler_params=pltpu.CompilerParams(use_tc_tiling_on_sc=False,
                                             needs_layout_passes=False),
        scratch_types=[
            pltpu.VMEM((CHD,), jnp.int32),
            pltpu.VMEM((CHD,), jnp.int32),
            pltpu.VMEM((CHD,), jnp.int32),
            pltpu.VMEM((CHD,), jnp.int32),
            pltpu.VMEM((CHD, D_TP_OUT), F32),
            pltpu.VMEM((CHD, D_TP_OUT), F32),
            pltpu.VMEM((CHD,), F32),
            pltpu.VMEM_SHARED((N, D_TP_OUT), F32),
            pltpu.VMEM_SHARED((N,), F32),
            pltpu.SemaphoreType.DMA,
            pltpu.SemaphoreType.DMA,
        ],
    )
    def _scatter_k(tp_hbm, dst_hbm, z8_hbm, z1_hbm, ones_hbm,
                   psum_hbm, pcnt_hbm,
                   idx0, idx1, t0, t1, rows0, rows1, ones_v, acc_sh, cnt_sh,
                   ls0, ls1):
        c = lax.axis_index("c")
        s = lax.axis_index("s")
        pltpu.sync_copy(z8_hbm.at[pl.ds(s * RT, RT)],
                        acc_sh.at[pl.ds(s * RT, RT)])

        @pl.when(s < 4)
        def _():
            pltpu.sync_copy(z1_hbm.at[pl.ds(s * CQ, CQ)],
                            cnt_sh.at[pl.ds(s * CQ, CQ)])

        pltpu.sync_copy(ones_hbm, ones_v)
        plsc.subcore_barrier()
        base = (s * 2 + c) * EW
        idx = (idx0, idx1)
        tid = (t0, t1)
        rows = (rows0, rows1)
        lsem = (ls0, ls1)
        lax.fori_loop(0, CHD // 16,
                      lambda i, _: (_tile_row_idx(i, base, tid[0]), 0)[1],
                      0, unroll=4)
        ld = {0: (pltpu.async_copy(dst_hbm.at[pl.ds(base, CHD)], idx[0],
                                   lsem[0]),
                  pltpu.async_copy(tp_hbm.at[tid[0]], rows[0],
                                   lsem[0]))}
        for k in range(NCHD):
            b = k % 2
            nb = (k + 1) % 2
            if k + 1 < NCHD:
                off = base + (k + 1) * CHD
                lax.fori_loop(
                    0, CHD // 16,
                    lambda i, _: (_tile_row_idx(i, off, tid[nb]), 0)[1],
                    0, unroll=4)
                ld[k + 1] = (
                    pltpu.async_copy(dst_hbm.at[pl.ds(off, CHD)], idx[nb],
                                     lsem[nb]),
                    pltpu.async_copy(tp_hbm.at[tid[nb]], rows[nb],
                                     lsem[nb]),
                )
            ld[k][0].wait()
            ld[k][1].wait()
            pltpu.sync_copy(rows[b], acc_sh.at[idx[b]], add=True)
            pltpu.sync_copy(ones_v, cnt_sh.at[idx[b]], add=True)
        plsc.subcore_barrier()
        pltpu.sync_copy(acc_sh.at[pl.ds(s * RT, RT)],
                        psum_hbm.at[c, pl.ds(s * RT, RT)])

        @pl.when(s < 4)
        def _():
            pltpu.sync_copy(cnt_sh.at[pl.ds(s * CQ, CQ)],
                            pcnt_hbm.at[c, pl.ds(s * CQ, CQ)])

    psum, pcnt = _scatter_k(tp_rows, dst, z8, z1, ones_h)

    big_wout = _blkdiag(W_lin_out * (1.0 / np.sqrt(D_TP_OUT)), 8)
    rcnt = np.zeros((8, 8 * D_OUT), np.float32)
    for j in range(8):
        rcnt[j, j * D_OUT:(j + 1) * D_OUT] = 1.0
    rcnt = jnp.asarray(rcnt)

    ps_p = psum.reshape(2, N // 8, 8 * D_TP_OUT)
    pc_p = pcnt.reshape(2, N // 8, 8)
    out_p = pl.pallas_call(
        _out_body,
        out_shape=jax.ShapeDtypeStruct((N // 8, 8 * D_OUT), F32),
    )(ps_p, pc_p, na_p, big_wout, rcnt)
    return out_p.reshape(N, D_OUT)
